# baseline (speedup 1.0000x reference)
import functools

import jax
import jax.numpy as jnp
from jax import lax
from jax.experimental import pallas as pl
from jax.experimental.pallas import tpu as pltpu
from jax.experimental.pallas import tpu_sc as plsc

_LANES = 16
_NT = 16
_NC = 2
_BLK = 1024


def _sc_deg(dstr, *, NP, R, CH):
    mesh = plsc.VectorSubcoreMesh(core_axis_name="c", subcore_axis_name="s")
    scratch = [
        pltpu.VMEM_SHARED((NP,), jnp.float32),
        pltpu.VMEM((CH, 128), jnp.int32),
        pltpu.VMEM((128,), jnp.float32),
        pltpu.VMEM((128,), jnp.float32),
        pltpu.VMEM((R,), jnp.float32),
        pltpu.VMEM((R, 128), jnp.float32),
    ]
    scratch += [pltpu.SemaphoreType.DMA for _ in range(_NBUF + 1)]

    @functools.partial(
        pl.kernel,
        out_type=jax.ShapeDtypeStruct((_NC, NP, 128), jnp.float32),
        mesh=mesh,
        scratch_types=scratch,
    )
    def k(dst_hbm, degb_hbm, sdeg, idxb, ones128, z128, degv, bcast, *sems):
        ssem = sems[:_NBUF]
        isem = sems[_NBUF]
        c = lax.axis_index("c")
        s = lax.axis_index("s")
        w = c * _NT + s
        nbase = s * R
        ic = pltpu.async_copy(dst_hbm.at[w], idxb, isem)
        for j in range(128 // _LANES):
            z128[pl.ds(_LANES * j, _LANES)] = jnp.zeros((_LANES,), jnp.float32)
            ones128[pl.ds(_LANES * j, _LANES)] = jnp.ones((_LANES,), jnp.float32)
        for j in range(R // 128):
            pltpu.sync_copy(z128, sdeg.at[pl.ds(nbase + 128 * j, 128)])
        ic.wait()
        plsc.subcore_barrier()

        def body(i, carry):
            sds = []
            for j in range(_NBUF):
                sds.append(
                    pltpu.async_copy(
                        ones128, sdeg.at[idxb.at[_NBUF * i + j]], ssem[j],
                        add=True,
                    )
                )
            for j in range(_NBUF):
                sds[j].wait()
            return carry

        lax.fori_loop(0, CH // _NBUF, body, 0)
        plsc.subcore_barrier()
        pltpu.sync_copy(sdeg.at[pl.ds(nbase, R)], degv)

        def bgrp(g, carry):
            v = degv[pl.ds(_LANES * g, _LANES)]
            for l in range(_LANES):
                row = jnp.zeros((_LANES,), jnp.float32) + v[l]
                brow = bcast.at[_LANES * g + l]
                for j in range(128 // _LANES):
                    brow[pl.ds(_LANES * j, _LANES)] = row
            return carry

        lax.fori_loop(0, R // _LANES, bgrp, 0)
        pltpu.sync_copy(bcast, degb_hbm.at[c, pl.ds(nbase, R)])

    return k(dstr)


_NBUF = 4
_GRP = 8


def _sc_agg(gt, srcr, dstr, *, NP, R, CH):
    mesh = plsc.VectorSubcoreMesh(core_axis_name="c", subcore_axis_name="s")
    G = CH // _GRP
    assert CH % _GRP == 0 and G % 2 == 0
    scratch = [
        pltpu.VMEM_SHARED((NP, 128), jnp.float32),
        pltpu.VMEM((_GRP, 128), jnp.int32),
        pltpu.VMEM((_GRP, 128), jnp.int32),
        pltpu.VMEM((_GRP, 128), jnp.int32),
        pltpu.VMEM((_GRP, 128), jnp.int32),
        pltpu.VMEM((128, 128), jnp.float32),
        pltpu.VMEM((128, 128), jnp.float32),
    ]
    scratch += [pltpu.SemaphoreType.DMA for _ in range(8)]

    @functools.partial(
        pl.kernel,
        out_type=jax.ShapeDtypeStruct((_NC, NP, 128), jnp.float32),
        mesh=mesh,
        scratch_types=scratch,
    )
    def k(gt_hbm, src_hbm, dst_hbm, accp_hbm, acc, sA, sB, dA, dB, buf0,
          buf1, gsem0, gsem1, ssem0, ssem1, iA0, iA1, iB0, iB1):
        bufs = (buf0, buf1)
        gsems = (gsem0, gsem1)
        ssems = (ssem0, ssem1)
        c = lax.axis_index("c")
        s = lax.axis_index("s")
        w = c * _NT + s
        nbase = s * R

        p0 = pltpu.async_copy(src_hbm.at[w, pl.ds(0, _GRP)], sA, iA0)
        p1 = pltpu.async_copy(dst_hbm.at[w, pl.ds(0, _GRP)], dA, iA1)

        def zrow(r, carry):
            zr = buf0.at[r]
            for j in range(128 // _LANES):
                zr[pl.ds(_LANES * j, _LANES)] = jnp.zeros((_LANES,), jnp.float32)
            return carry

        lax.fori_loop(0, 128, zrow, 0)
        for j in range(R // 128):
            pltpu.sync_copy(buf0, acc.at[pl.ds(nbase + 128 * j, 128)])
        p0.wait()
        p1.wait()
        plsc.subcore_barrier()

        def process(si, di):
            gds = [None] * _GRP
            sds = [None] * _GRP
            gds[0] = pltpu.async_copy(gt_hbm.at[si.at[0]], bufs[0], gsems[0])
            gds[1] = pltpu.async_copy(gt_hbm.at[si.at[1]], bufs[1], gsems[1])
            for j in range(_GRP):
                b = j % 2
                gds[j].wait()
                sds[j] = pltpu.async_copy(
                    bufs[b], acc.at[di.at[j]], ssems[b], add=True
                )
                if j + 2 < _GRP:
                    sds[j].wait()
                    gds[j + 2] = pltpu.async_copy(
                        gt_hbm.at[si.at[j + 2]], bufs[b], gsems[b]
                    )
            sds[_GRP - 2].wait()
            sds[_GRP - 1].wait()

        def pair(p, carry):
            qB = pltpu.async_copy(
                src_hbm.at[w, pl.ds(_GRP * (2 * p + 1), _GRP)], sB, iB0
            )
            qB1 = pltpu.async_copy(
                dst_hbm.at[w, pl.ds(_GRP * (2 * p + 1), _GRP)], dB, iB1
            )
            process(sA, dA)
            qB.wait()
            qB1.wait()
            nxt = jnp.minimum(_GRP * (2 * p + 2), _GRP * (G - 1))
            qA = pltpu.async_copy(src_hbm.at[w, pl.ds(nxt, _GRP)], sA, iA0)
            qA1 = pltpu.async_copy(dst_hbm.at[w, pl.ds(nxt, _GRP)], dA, iA1)
            process(sB, dB)
            qA.wait()
            qA1.wait()
            return carry

        lax.fori_loop(0, G // 2, pair, 0)
        plsc.subcore_barrier()
        pltpu.sync_copy(acc.at[pl.ds(nbase, R)], accp_hbm.at[c, pl.ds(nbase, R)])

    return k(gt, srcr, dstr)


def _row_specs(np_, d, n):
    return [pl.BlockSpec((_BLK, d), lambda r: (r, 0)) for _ in range(n)]


def _tc_matmul0(xp, W, br):
    NP, D = xp.shape
    H = W.shape[1]

    def body(x_ref, w_ref, b_ref, o_ref):
        o_ref[...] = (
            jnp.dot(x_ref[...], w_ref[...], preferred_element_type=jnp.float32)
            + b_ref[...]
        )

    return pl.pallas_call(
        body,
        grid=(NP // _BLK,),
        in_specs=[
            pl.BlockSpec((_BLK, D), lambda r: (r, 0)),
            pl.BlockSpec((D, H), lambda r: (0, 0)),
            pl.BlockSpec((1, H), lambda r: (0, 0)),
        ],
        out_specs=pl.BlockSpec((_BLK, H), lambda r: (r, 0)),
        out_shape=jax.ShapeDtypeStruct((NP, H), jnp.float32),
    )(xp, W, br)


def _tc_scale(degb, g0):
    _, NP, D = degb.shape

    def body(d_ref, g_ref, dinv_ref, gt_ref):
        d = d_ref[...]
        dinv = lax.rsqrt(d[0] + d[1] + 1.0)
        dinv_ref[...] = dinv
        gt_ref[...] = g_ref[...] * dinv

    return pl.pallas_call(
        body,
        grid=(NP // _BLK,),
        in_specs=[
            pl.BlockSpec((_NC, _BLK, D), lambda r: (0, r, 0)),
            pl.BlockSpec((_BLK, D), lambda r: (r, 0)),
        ],
        out_specs=[
            pl.BlockSpec((_BLK, D), lambda r: (r, 0)),
            pl.BlockSpec((_BLK, D), lambda r: (r, 0)),
        ],
        out_shape=[
            jax.ShapeDtypeStruct((NP, D), jnp.float32),
            jax.ShapeDtypeStruct((NP, D), jnp.float32),
        ],
    )(degb, g0)


def _tc_layer(accp, gt0, dinvb, W, br):
    _, NP, D = accp.shape
    H = W.shape[1]

    def body(a_ref, gt_ref, dv_ref, w_ref, b_ref, h1_ref, gt1_ref):
        a = a_ref[...]
        dv = dv_ref[...]
        h1 = jnp.maximum(dv * (a[0] + a[1] + gt_ref[...]), 0.0)
        h1_ref[...] = h1
        gt1_ref[...] = (
            jnp.dot(h1, w_ref[...], preferred_element_type=jnp.float32)
            + b_ref[...]
        ) * dv

    return pl.pallas_call(
        body,
        grid=(NP // _BLK,),
        in_specs=[
            pl.BlockSpec((_NC, _BLK, D), lambda r: (0, r, 0)),
            pl.BlockSpec((_BLK, D), lambda r: (r, 0)),
            pl.BlockSpec((_BLK, D), lambda r: (r, 0)),
            pl.BlockSpec((D, H), lambda r: (0, 0)),
            pl.BlockSpec((1, H), lambda r: (0, 0)),
        ],
        out_specs=[
            pl.BlockSpec((_BLK, D), lambda r: (r, 0)),
            pl.BlockSpec((_BLK, H), lambda r: (r, 0)),
        ],
        out_shape=[
            jax.ShapeDtypeStruct((NP, D), jnp.float32),
            jax.ShapeDtypeStruct((NP, H), jnp.float32),
        ],
    )(accp, gt0, dinvb, W, br)


def _tc_final(accp, gt1, dinvb, h1, WoutP, boutP):
    _, NP, D = accp.shape

    def body(a_ref, gt_ref, dv_ref, h1_ref, w_ref, b_ref, o_ref):
        a = a_ref[...]
        h2 = jnp.maximum(dv_ref[...] * (a[0] + a[1] + gt_ref[...]), 0.0)
        h1 = h1_ref[...]
        s1 = jnp.sum(h1 * h1, axis=1, keepdims=True)
        s2 = jnp.sum(h2 * h2, axis=1, keepdims=True)
        m = jnp.maximum(s1, s2)
        e1 = jnp.exp(s1 - m)
        e2 = jnp.exp(s2 - m)
        h = (e1 * h1 + e2 * h2) / (e1 + e2)
        o_ref[...] = (
            jnp.dot(h, w_ref[...], preferred_element_type=jnp.float32)
            + b_ref[...]
        )

    return pl.pallas_call(
        body,
        grid=(NP // _BLK,),
        in_specs=[
            pl.BlockSpec((_NC, _BLK, D), lambda r: (0, r, 0)),
            pl.BlockSpec((_BLK, D), lambda r: (r, 0)),
            pl.BlockSpec((_BLK, D), lambda r: (r, 0)),
            pl.BlockSpec((_BLK, D), lambda r: (r, 0)),
            pl.BlockSpec((D, D), lambda r: (0, 0)),
            pl.BlockSpec((1, D), lambda r: (0, 0)),
        ],
        out_specs=pl.BlockSpec((_BLK, D), lambda r: (r, 0)),
        out_shape=jax.ShapeDtypeStruct((NP, D), jnp.float32),
    )(accp, gt1, dinvb, h1, WoutP, boutP)


def kernel(x, edge_index, W0, b0, W1, b1, Wout, bout):
    N, D = x.shape
    E = edge_index.shape[1]
    H = W0.shape[1]
    C = Wout.shape[1]

    R = -(-N // _NT)
    R = -(-R // 128) * 128
    NP = _NT * R
    NW = _NC * _NT
    CH = -(-E // (NW * 128 * 2 * _GRP)) * (2 * _GRP)
    EP = NW * CH * 128

    xp = jnp.pad(x, ((0, NP - N), (0, 0)))
    src = edge_index[0].astype(jnp.int32)
    dst = edge_index[1].astype(jnp.int32)
    srcr = jnp.pad(src, (0, EP - E), constant_values=N).reshape(NW, CH, 128)
    dstr = jnp.pad(dst, (0, EP - E), constant_values=N).reshape(NW, CH, 128)
    b0r = b0.reshape(1, H)
    b1r = b1.reshape(1, H)
    WoutP = jnp.pad(Wout, ((0, 0), (0, D - C)))
    boutP = jnp.pad(bout, (0, D - C)).reshape(1, D)

    degb = _sc_deg(dstr, NP=NP, R=R, CH=CH)
    g0 = _tc_matmul0(xp, W0, b0r)
    dinvb, gt0 = _tc_scale(degb, g0)
    accp1 = _sc_agg(gt0, srcr, dstr, NP=NP, R=R, CH=CH)
    h1, gt1 = _tc_layer(accp1, gt0, dinvb, W1, b1r)
    accp2 = _sc_agg(gt1, srcr, dstr, NP=NP, R=R, CH=CH)
    outp = _tc_final(accp2, gt1, dinvb, h1, WoutP, boutP)
    return outp[:N, :C]
